# Initial kernel scaffold; baseline (speedup 1.0000x reference)
import functools
import math

import jax
import jax.numpy as jnp
from jax import lax
from jax.experimental import pallas as pl
from jax.experimental.pallas import tpu as pltpu
from jax.experimental.pallas import tpu_sc as plsc

EMB = 32
SCALE = math.sqrt(float(EMB))
NC = 2
NS = 16
NW = NC * NS
L = 16
CHUNK = 128


def _sc_embed(tokens2d, table):
    n_idx_rows = tokens2d.shape[0]
    total = n_idx_rows * CHUNK
    rows_per_w = n_idx_rows // NW
    mesh = plsc.VectorSubcoreMesh(core_axis_name="c", subcore_axis_name="s")

    @functools.partial(
        pl.kernel,
        mesh=mesh,
        out_type=jax.ShapeDtypeStruct((total, EMB), jnp.float32),
        scratch_types=[
            pltpu.VMEM((rows_per_w, CHUNK), jnp.int32),
            pltpu.VMEM((CHUNK, EMB), jnp.float32),
            pltpu.SemaphoreType.DMA,
        ],
    )
    def k(tok_hbm, tab_hbm, out_hbm, idx_v, rows_v, sem):
        wid = lax.axis_index("s") * NC + lax.axis_index("c")
        row0 = wid * rows_per_w
        pltpu.sync_copy(tok_hbm.at[pl.ds(row0, rows_per_w)], idx_v)

        def chunk_body(g, carry):
            pltpu.async_copy(tab_hbm.at[idx_v.at[g]], rows_v, sem).wait()

            def scale_body(i, c):
                rows_v[i, pl.ds(0, L)] = rows_v[i, pl.ds(0, L)] * SCALE
                rows_v[i, pl.ds(L, L)] = rows_v[i, pl.ds(L, L)] * SCALE
                return c

            lax.fori_loop(0, CHUNK, scale_body, 0, unroll=4)
            pltpu.sync_copy(
                rows_v, out_hbm.at[pl.ds((row0 + g) * CHUNK, CHUNK)]
            )
            return carry

        lax.fori_loop(0, rows_per_w, chunk_body, 0)

    return k(tokens2d, table)


def kernel(tokens, embedding_weight):
    b, s = tokens.shape
    flat = tokens.astype(jnp.int32).reshape(-1, CHUNK)
    out = _sc_embed(flat, embedding_weight)
    return out.reshape(b, s, EMB)


# speedup vs baseline: 1.2571x; 1.2571x over previous
import functools
import math

import jax
import jax.numpy as jnp
from jax import lax
from jax.experimental import pallas as pl
from jax.experimental.pallas import tpu as pltpu
from jax.experimental.pallas import tpu_sc as plsc

EMB = 32
SCALE = math.sqrt(float(EMB))
NC = 2
NS = 16
NW = NC * NS
L = 16
CHUNK = 128


def _sc_embed(tokens2d, table):
    n_idx_rows = tokens2d.shape[0]
    total = n_idx_rows * CHUNK
    rows_per_w = n_idx_rows // NW
    mesh = plsc.VectorSubcoreMesh(core_axis_name="c", subcore_axis_name="s")

    @functools.partial(
        pl.kernel,
        mesh=mesh,
        compiler_params=pltpu.CompilerParams(use_tc_tiling_on_sc=False),
        out_type=jax.ShapeDtypeStruct((total, EMB), jnp.float32),
        scratch_types=[
            pltpu.VMEM((rows_per_w, CHUNK), jnp.int32),
            pltpu.VMEM((CHUNK, EMB), jnp.float32),
            pltpu.SemaphoreType.DMA,
        ],
    )
    def k(tok_hbm, tab_hbm, out_hbm, idx_v, rows_v, sem):
        wid = lax.axis_index("s") * NC + lax.axis_index("c")
        row0 = wid * rows_per_w
        pltpu.sync_copy(tok_hbm.at[pl.ds(row0, rows_per_w)], idx_v)

        def chunk_body(g, carry):
            pltpu.async_copy(tab_hbm.at[idx_v.at[g]], rows_v, sem).wait()

            def scale_body(i, c):
                rows_v[i, pl.ds(0, L)] = rows_v[i, pl.ds(0, L)] * SCALE
                rows_v[i, pl.ds(L, L)] = rows_v[i, pl.ds(L, L)] * SCALE
                return c

            lax.fori_loop(0, CHUNK, scale_body, 0, unroll=4)
            pltpu.sync_copy(
                rows_v, out_hbm.at[pl.ds((row0 + g) * CHUNK, CHUNK)]
            )
            return carry

        lax.fori_loop(0, rows_per_w, chunk_body, 0)

    return k(tokens2d, table)


def kernel(tokens, embedding_weight):
    b, s = tokens.shape
    flat = tokens.astype(jnp.int32).reshape(-1, CHUNK)
    out = _sc_embed(flat, embedding_weight)
    return out.reshape(b, s, EMB)


# speedup vs baseline: 1.2664x; 1.0074x over previous
import functools
import math

import jax
import jax.numpy as jnp
from jax import lax
from jax.experimental import pallas as pl
from jax.experimental.pallas import tpu as pltpu
from jax.experimental.pallas import tpu_sc as plsc

EMB = 32
SCALE = math.sqrt(float(EMB))
NC = 2
NS = 16
NW = NC * NS
L = 16
CHUNK = 128
NBUF = 4
NOBUF = 2


def _sc_embed(tokens2d, table):
    n_idx_rows = tokens2d.shape[0]
    total = n_idx_rows * CHUNK
    rows_per_w = n_idx_rows // NW
    n_outer = rows_per_w // NBUF
    mesh = plsc.VectorSubcoreMesh(core_axis_name="c", subcore_axis_name="s")

    @functools.partial(
        pl.kernel,
        mesh=mesh,
        compiler_params=pltpu.CompilerParams(use_tc_tiling_on_sc=False),
        out_type=jax.ShapeDtypeStruct((total, EMB), jnp.float32),
        scratch_types=[
            pltpu.VMEM((rows_per_w, CHUNK), jnp.int32),
            pltpu.VMEM((NBUF, CHUNK, EMB), jnp.float32),
            pltpu.VMEM((NOBUF, CHUNK, EMB), jnp.float32),
            pltpu.SemaphoreType.DMA((NBUF,)),
            pltpu.SemaphoreType.DMA((NOBUF,)),
        ],
    )
    def k(tok_hbm, tab_hbm, out_hbm, idx_v, rows_v, obuf_v, sem_g, sem_o):
        wid = lax.axis_index("s") * NC + lax.axis_index("c")
        row0 = wid * rows_per_w
        pltpu.sync_copy(tok_hbm.at[pl.ds(row0, rows_per_w)], idx_v)

        for b in range(NBUF):
            pltpu.async_copy(tab_hbm.at[idx_v.at[b]], rows_v.at[b], sem_g.at[b])

        def outer(i, carry):
            for b in range(NBUF):
                c = i * NBUF + b
                ob = b % NOBUF
                if b < NOBUF:
                    @pl.when(i > 0)
                    def _():
                        pltpu.make_async_copy(
                            obuf_v.at[ob],
                            out_hbm.at[pl.ds((row0 + c - NOBUF) * CHUNK, CHUNK)],
                            sem_o.at[ob],
                        ).wait()
                else:
                    pltpu.make_async_copy(
                        obuf_v.at[ob],
                        out_hbm.at[pl.ds((row0 + c - NOBUF) * CHUNK, CHUNK)],
                        sem_o.at[ob],
                    ).wait()
                pltpu.make_async_copy(
                    tab_hbm.at[idx_v.at[c]], rows_v.at[b], sem_g.at[b]
                ).wait()

                def scale_body(r, cc):
                    obuf_v[ob, r, pl.ds(0, L)] = rows_v[b, r, pl.ds(0, L)] * SCALE
                    obuf_v[ob, r, pl.ds(L, L)] = rows_v[b, r, pl.ds(L, L)] * SCALE
                    return cc

                lax.fori_loop(0, CHUNK, scale_body, 0, unroll=8)

                pltpu.async_copy(
                    obuf_v.at[ob],
                    out_hbm.at[pl.ds((row0 + c) * CHUNK, CHUNK)],
                    sem_o.at[ob],
                )

                @pl.when(i < n_outer - 1)
                def _():
                    pltpu.async_copy(
                        tab_hbm.at[idx_v.at[c + NBUF]],
                        rows_v.at[b],
                        sem_g.at[b],
                    )
            return carry

        lax.fori_loop(0, n_outer, outer, 0)

        for ob in range(NOBUF):
            c = rows_per_w - NOBUF + ob
            pltpu.make_async_copy(
                obuf_v.at[ob],
                out_hbm.at[pl.ds((row0 + c) * CHUNK, CHUNK)],
                sem_o.at[ob],
            ).wait()

    return k(tokens2d, table)


def kernel(tokens, embedding_weight):
    b, s = tokens.shape
    flat = tokens.astype(jnp.int32).reshape(-1, CHUNK)
    out = _sc_embed(flat, embedding_weight)
    return out.reshape(b, s, EMB)


# speedup vs baseline: 1.4385x; 1.1359x over previous
import functools
import math

import jax
import jax.numpy as jnp
from jax import lax
from jax.experimental import pallas as pl
from jax.experimental.pallas import tpu as pltpu
from jax.experimental.pallas import tpu_sc as plsc

EMB = 32
SCALE = math.sqrt(float(EMB))
NC = 2
NS = 16
NW = NC * NS
L = 16
BBLK = 128


def _sc_embed(tok_native, table, n_seq, n_batch):
    sgrp = n_seq // 8
    nblk = n_batch // BBLK
    mesh = plsc.VectorSubcoreMesh(core_axis_name="c", subcore_axis_name="s")

    @functools.partial(
        pl.kernel,
        mesh=mesh,
        compiler_params=pltpu.CompilerParams(
            use_tc_tiling_on_sc=False, needs_layout_passes=False
        ),
        out_type=jax.ShapeDtypeStruct(
            (n_seq, EMB // 8, nblk, 8, BBLK), jnp.float32
        ),
        scratch_types=[
            pltpu.VMEM((sgrp, 8, BBLK), jnp.int32),
            pltpu.VMEM((2, BBLK, EMB), jnp.float32),
            pltpu.VMEM((2, EMB // 8, 8, BBLK), jnp.float32),
            pltpu.SemaphoreType.DMA((2,)),
            pltpu.SemaphoreType.DMA((2,)),
        ],
    )
    def k(tok_hbm, tab_hbm, out_hbm, idx_v, rows_v, trans_v, sem_g, sem_o):
        wid = lax.axis_index("s") * NC + lax.axis_index("c")
        pltpu.sync_copy(tok_hbm.at[:, wid], idx_v)

        iota = lax.iota(jnp.int32, L)
        e_hi = iota // 8
        e_lo = iota % 8

        def start_gather(s, b):
            pltpu.async_copy(
                tab_hbm.at[idx_v.at[s // 8, s % 8]], rows_v.at[b], sem_g.at[b]
            )

        def wait_gather(s, b):
            pltpu.make_async_copy(
                tab_hbm.at[idx_v.at[s // 8, s % 8]], rows_v.at[b], sem_g.at[b]
            ).wait()

        def start_out(s, b):
            pltpu.async_copy(
                trans_v.at[b], out_hbm.at[s, :, wid], sem_o.at[b]
            )

        def wait_out(s, b):
            pltpu.make_async_copy(
                trans_v.at[b], out_hbm.at[s, :, wid], sem_o.at[b]
            ).wait()

        start_gather(0, 0)
        start_gather(1, 1)

        def outer(i, carry):
            for b in range(2):
                s = 2 * i + b
                @pl.when(i > 0)
                def _():
                    wait_out(s - 2, b)

                wait_gather(s, b)

                def token_body(t, cc):
                    lane = jnp.full((L,), 0, jnp.int32) + t
                    v0 = rows_v[b, t, pl.ds(0, L)] * SCALE
                    v1 = rows_v[b, t, pl.ds(L, L)] * SCALE
                    plsc.store_scatter(trans_v.at[b], [e_hi, e_lo, lane], v0)
                    plsc.store_scatter(
                        trans_v.at[b], [e_hi + 2, e_lo, lane], v1
                    )
                    return cc

                lax.fori_loop(0, BBLK, token_body, 0, unroll=4)

                start_out(s, b)

                @pl.when(s + 2 < 2 * n_outer)
                def _():
                    start_gather(s + 2, b)
            return carry

        n_outer = (sgrp * 8) // 2
        lax.fori_loop(0, n_outer, outer, 0)
        wait_out(2 * n_outer - 2, 0)
        wait_out(2 * n_outer - 1, 1)

    return k(tok_native, table)


def kernel(tokens, embedding_weight):
    nb, ns = tokens.shape
    tok_native = (
        tokens.astype(jnp.int32)
        .T.reshape(ns // 8, 8, nb // BBLK, BBLK)
        .transpose(0, 2, 1, 3)
    )
    out5 = _sc_embed(tok_native, embedding_weight, ns, nb)
    return out5.transpose(2, 4, 0, 1, 3).reshape(nb, ns, EMB)


# speedup vs baseline: 2.1702x; 1.5086x over previous
import functools
import math

import jax
import jax.numpy as jnp
from jax import lax
from jax.experimental import pallas as pl
from jax.experimental.pallas import tpu as pltpu
from jax.experimental.pallas import tpu_sc as plsc

EMB = 32
SCALE = math.sqrt(float(EMB))
NC = 2
NS = 16
NW = NC * NS
L = 16
BBLK = 128


def _sc_embed(tok_native, table, n_seq, n_batch):
    sgrp = n_seq // 8
    nblk = n_batch // BBLK
    mesh = plsc.VectorSubcoreMesh(core_axis_name="c", subcore_axis_name="s")

    @functools.partial(
        pl.kernel,
        mesh=mesh,
        compiler_params=pltpu.CompilerParams(
            use_tc_tiling_on_sc=False, needs_layout_passes=False
        ),
        out_type=jax.ShapeDtypeStruct(
            (n_seq, EMB // 8, nblk, 8, BBLK), jnp.float32
        ),
        scratch_types=[
            pltpu.VMEM((sgrp, 8, BBLK), jnp.int32),
            pltpu.VMEM((2, BBLK, EMB), jnp.float32),
            pltpu.VMEM((2, EMB // 8, 8, BBLK + 1), jnp.float32),
            pltpu.SemaphoreType.DMA((2,)),
            pltpu.SemaphoreType.DMA((2,)),
        ],
    )
    def k(tok_hbm, tab_hbm, out_hbm, idx_v, rows_v, trans_v, sem_g, sem_o):
        wid = lax.axis_index("s") * NC + lax.axis_index("c")
        pltpu.sync_copy(tok_hbm.at[:, wid], idx_v)

        iota = lax.iota(jnp.int32, L)
        e_hi = iota // 8
        e_lo = iota % 8

        def start_gather(s, b):
            pltpu.async_copy(
                tab_hbm.at[idx_v.at[s // 8, s % 8]], rows_v.at[b], sem_g.at[b]
            )

        def wait_gather(s, b):
            pltpu.make_async_copy(
                tab_hbm.at[idx_v.at[s // 8, s % 8]], rows_v.at[b], sem_g.at[b]
            ).wait()

        def start_out(s, b):
            pltpu.async_copy(
                trans_v.at[b, :, :, pl.ds(0, BBLK)],
                out_hbm.at[s, :, wid],
                sem_o.at[b],
            )

        def wait_out(s, b):
            pltpu.make_async_copy(
                trans_v.at[b, :, :, pl.ds(0, BBLK)],
                out_hbm.at[s, :, wid],
                sem_o.at[b],
            ).wait()

        start_gather(0, 0)
        start_gather(1, 1)

        def outer(i, carry):
            for b in range(2):
                s = 2 * i + b
                @pl.when(i > 0)
                def _():
                    wait_out(s - 2, b)

                wait_gather(s, b)

                def token_body(t, cc):
                    lane = jnp.full((L,), 0, jnp.int32) + t
                    v0 = rows_v[b, t, pl.ds(0, L)] * SCALE
                    v1 = rows_v[b, t, pl.ds(L, L)] * SCALE
                    plsc.store_scatter(trans_v.at[b], [e_hi, e_lo, lane], v0)
                    plsc.store_scatter(
                        trans_v.at[b], [e_hi + 2, e_lo, lane], v1
                    )
                    return cc

                lax.fori_loop(0, BBLK, token_body, 0, unroll=8)

                start_out(s, b)

                @pl.when(s + 2 < 2 * n_outer)
                def _():
                    start_gather(s + 2, b)
            return carry

        n_outer = (sgrp * 8) // 2
        lax.fori_loop(0, n_outer, outer, 0)
        wait_out(2 * n_outer - 2, 0)
        wait_out(2 * n_outer - 1, 1)

    return k(tok_native, table)


def kernel(tokens, embedding_weight):
    nb, ns = tokens.shape
    tok_native = (
        tokens.astype(jnp.int32)
        .T.reshape(ns // 8, 8, nb // BBLK, BBLK)
        .transpose(0, 2, 1, 3)
    )
    out5 = _sc_embed(tok_native, embedding_weight, ns, nb)
    return out5.transpose(2, 4, 0, 1, 3).reshape(nb, ns, EMB)


# speedup vs baseline: 2.2408x; 1.0325x over previous
import functools
import math

import jax
import jax.numpy as jnp
from jax import lax
from jax.experimental import pallas as pl
from jax.experimental.pallas import tpu as pltpu
from jax.experimental.pallas import tpu_sc as plsc

EMB = 32
SCALE = math.sqrt(float(EMB))
NC = 2
NS = 16
NW = NC * NS
L = 16
BBLK = 128
NBUF = 4


def _sc_embed(tok_native, table, n_seq, n_batch):
    sgrp = n_seq // 8
    nblk = n_batch // BBLK
    mesh = plsc.VectorSubcoreMesh(core_axis_name="c", subcore_axis_name="s")

    @functools.partial(
        pl.kernel,
        mesh=mesh,
        compiler_params=pltpu.CompilerParams(
            use_tc_tiling_on_sc=False, needs_layout_passes=False
        ),
        out_type=jax.ShapeDtypeStruct(
            (n_seq, EMB // 8, nblk, 8, BBLK), jnp.float32
        ),
        scratch_types=[
            pltpu.VMEM((sgrp, 8, BBLK), jnp.int32),
            pltpu.VMEM((NBUF, BBLK, EMB), jnp.float32),
            pltpu.VMEM((NBUF, EMB // 8, 8, BBLK + 1), jnp.float32),
            pltpu.SemaphoreType.DMA((NBUF,)),
            pltpu.SemaphoreType.DMA((NBUF,)),
        ],
    )
    def k(tok_hbm, tab_hbm, out_hbm, idx_v, rows_v, trans_v, sem_g, sem_o):
        wid = lax.axis_index("s") * NC + lax.axis_index("c")
        pltpu.sync_copy(tok_hbm.at[:, wid], idx_v)

        iota = lax.iota(jnp.int32, L)
        e_hi = iota // 8
        e_lo = iota % 8

        def start_gather(s, b):
            pltpu.async_copy(
                tab_hbm.at[idx_v.at[s // 8, s % 8]], rows_v.at[b], sem_g.at[b]
            )

        def wait_gather(s, b):
            pltpu.make_async_copy(
                tab_hbm.at[idx_v.at[s // 8, s % 8]], rows_v.at[b], sem_g.at[b]
            ).wait()

        def start_out(s, b):
            pltpu.async_copy(
                trans_v.at[b, :, :, pl.ds(0, BBLK)],
                out_hbm.at[s, :, wid],
                sem_o.at[b],
            )

        def wait_out(s, b):
            pltpu.make_async_copy(
                trans_v.at[b, :, :, pl.ds(0, BBLK)],
                out_hbm.at[s, :, wid],
                sem_o.at[b],
            ).wait()

        for b in range(NBUF):
            start_gather(b, b)

        def outer(i, carry):
            for b in range(NBUF):
                s = NBUF * i + b
                @pl.when(i > 0)
                def _():
                    wait_out(s - NBUF, b)

                wait_gather(s, b)

                def token_body(t, cc):
                    lane = jnp.full((L,), 0, jnp.int32) + t
                    v0 = rows_v[b, t, pl.ds(0, L)] * SCALE
                    v1 = rows_v[b, t, pl.ds(L, L)] * SCALE
                    plsc.store_scatter(trans_v.at[b], [e_hi, e_lo, lane], v0)
                    plsc.store_scatter(
                        trans_v.at[b], [e_hi + 2, e_lo, lane], v1
                    )
                    return cc

                lax.fori_loop(0, BBLK, token_body, 0, unroll=8)

                start_out(s, b)

                @pl.when(s + NBUF < NBUF * n_outer)
                def _():
                    start_gather(s + NBUF, b)
            return carry

        n_outer = (sgrp * 8) // NBUF
        lax.fori_loop(0, n_outer, outer, 0)
        for b in range(NBUF):
            wait_out(NBUF * n_outer - NBUF + b, b)

    return k(tok_native, table)


def kernel(tokens, embedding_weight):
    nb, ns = tokens.shape
    tok_native = (
        tokens.astype(jnp.int32)
        .T.reshape(ns // 8, 8, nb // BBLK, BBLK)
        .transpose(0, 2, 1, 3)
    )
    out5 = _sc_embed(tok_native, embedding_weight, ns, nb)
    return out5.transpose(2, 4, 0, 1, 3).reshape(nb, ns, EMB)
